# Initial kernel scaffold; baseline (speedup 1.0000x reference)
import functools
import math

import jax
import jax.numpy as jnp
from jax import lax
from jax.experimental import pallas as pl
from jax.experimental.pallas import tpu as pltpu
from jax.experimental.pallas import tpu_sc as plsc

_N = 10000
_NPAD = 10240
_E = 160000
_EPAD = 163840
_NB = 128
_EBLK = 40
_PAD_NODE = 10200
_D = 128
_H = 256
_BLK = 256
_NC = 2
_NS = 16
_ROWS_PER_TILE = _NPAD // _NS


def _rs(x):
    return lax.rsqrt(jnp.maximum(x, 1.0))


def _enc_body(x_ref, w1_ref, d1_ref, w2_ref, d2_ref, o_ref):
    h = jnp.dot(x_ref[...], w1_ref[...], preferred_element_type=jnp.float32)
    h = jnp.maximum(h + d1_ref[...], 0.0)
    o = jnp.dot(h, w2_ref[...], preferred_element_type=jnp.float32)
    o_ref[...] = o + d2_ref[...]


def _encode(x, w1f, d1, w2f, d2):
    din = x.shape[1]
    return pl.pallas_call(
        _enc_body,
        grid=(_NPAD // _BLK,),
        in_specs=[
            pl.BlockSpec((_BLK, din), lambda i: (i, 0)),
            pl.BlockSpec((din, _H), lambda i: (0, 0)),
            pl.BlockSpec((1, _H), lambda i: (0, 0)),
            pl.BlockSpec((_H, _D), lambda i: (0, 0)),
            pl.BlockSpec((1, _D), lambda i: (0, 0)),
        ],
        out_specs=pl.BlockSpec((_BLK, _D), lambda i: (i, 0)),
        out_shape=jax.ShapeDtypeStruct((_NPAD, _D), jnp.float32),
    )(x, w1f, d1, w2f, d2)


def _xw1_body(ha, hb, f_ref, wa_ref, wb_ref, deg_ref, ya_ref, yb_ref):
    f = f_ref[...]
    sa = _rs(deg_ref[:, ha:ha + 1])
    sb = _rs(deg_ref[:, hb:hb + 1])
    ya_ref[...] = jnp.dot(f, wa_ref[...], preferred_element_type=jnp.float32) * sa
    yb_ref[...] = jnp.dot(f, wb_ref[...], preferred_element_type=jnp.float32) * sb


def _xw1(f, wa, wb, deg, ha, hb):
    return pl.pallas_call(
        functools.partial(_xw1_body, ha, hb),
        grid=(_NPAD // _BLK,),
        in_specs=[
            pl.BlockSpec((_BLK, _D), lambda i: (i, 0)),
            pl.BlockSpec((_D, _D), lambda i: (0, 0)),
            pl.BlockSpec((_D, _D), lambda i: (0, 0)),
            pl.BlockSpec((_BLK, 16), lambda i: (i, 0)),
        ],
        out_specs=[
            pl.BlockSpec((_BLK, _D), lambda i: (i, 0)),
            pl.BlockSpec((_BLK, _D), lambda i: (i, 0)),
        ],
        out_shape=[
            jax.ShapeDtypeStruct((_NPAD, _D), jnp.float32),
            jax.ShapeDtypeStruct((_NPAD, _D), jnp.float32),
        ],
    )(f, wa, wb, deg)


def _xw2_body(hdA, hdB, ha, hb, pA_ref, pB_ref, deg_ref, bA_ref, bB_ref,
              wa_ref, wb_ref, ya_ref, yb_ref):
    gA = (pA_ref[0] + pA_ref[1]) * _rs(deg_ref[:, hdA:hdA + 1]) + bA_ref[...]
    gB = (pB_ref[0] + pB_ref[1]) * _rs(deg_ref[:, hdB:hdB + 1]) + bB_ref[...]
    f = jnp.maximum(gA + gB, 0.0)
    sa = _rs(deg_ref[:, ha:ha + 1])
    sb = _rs(deg_ref[:, hb:hb + 1])
    ya_ref[...] = jnp.dot(f, wa_ref[...], preferred_element_type=jnp.float32) * sa
    yb_ref[...] = jnp.dot(f, wb_ref[...], preferred_element_type=jnp.float32) * sb


def _xw2(pA, pB, deg, hdA, hdB, bA, bB, wa, wb, ha, hb):
    return pl.pallas_call(
        functools.partial(_xw2_body, hdA, hdB, ha, hb),
        grid=(_NPAD // _BLK,),
        in_specs=[
            pl.BlockSpec((_NC, _BLK, _D), lambda i: (0, i, 0)),
            pl.BlockSpec((_NC, _BLK, _D), lambda i: (0, i, 0)),
            pl.BlockSpec((_BLK, 16), lambda i: (i, 0)),
            pl.BlockSpec((1, _D), lambda i: (0, 0)),
            pl.BlockSpec((1, _D), lambda i: (0, 0)),
            pl.BlockSpec((_D, _D), lambda i: (0, 0)),
            pl.BlockSpec((_D, _D), lambda i: (0, 0)),
        ],
        out_specs=[
            pl.BlockSpec((_BLK, _D), lambda i: (i, 0)),
            pl.BlockSpec((_BLK, _D), lambda i: (i, 0)),
        ],
        out_shape=[
            jax.ShapeDtypeStruct((_NPAD, _D), jnp.float32),
            jax.ShapeDtypeStruct((_NPAD, _D), jnp.float32),
        ],
    )(pA, pB, deg, bA, bB, wa, wb)


def _readout_body(p0_ref, p1_ref, p2_ref, p3_ref, deg_ref,
                  b0_ref, b1_ref, b2_ref, b3_ref,
                  wp1_ref, bp1_ref, wp2_ref, bp2_ref, o_ref, s_ref):
    i = pl.program_id(0)
    fg = jnp.maximum(
        (p0_ref[0] + p0_ref[1]) * _rs(deg_ref[:, 1:2]) + b0_ref[...]
        + (p2_ref[0] + p2_ref[1]) * _rs(deg_ref[:, 5:6]) + b2_ref[...], 0.0)
    fp = jnp.maximum(
        (p1_ref[0] + p1_ref[1]) * _rs(deg_ref[:, 3:4]) + b1_ref[...]
        + (p3_ref[0] + p3_ref[1]) * _rs(deg_ref[:, 7:8]) + b3_ref[...], 0.0)
    rows = i * _BLK + lax.broadcasted_iota(jnp.int32, (_BLK, 1), 0)
    contrib = jnp.where(rows < _N, fg + fp, 0.0)

    @pl.when(i == 0)
    def _():
        s_ref[...] = jnp.zeros_like(s_ref)

    s_ref[...] += contrib

    @pl.when(i == _NPAD // _BLK - 1)
    def _():
        g = jnp.sum(s_ref[...], axis=0, keepdims=True) * (1.0 / (2 * _N))
        h = jnp.maximum(
            jnp.dot(g, wp1_ref[...], preferred_element_type=jnp.float32)
            + bp1_ref[...], 0.0)
        out = jnp.dot(h, wp2_ref[...], preferred_element_type=jnp.float32)
        o_ref[...] = jnp.broadcast_to(out + bp2_ref[...], (8, _D))


def _readout(p0, p1, p2, p3, deg, b0, b1, b2, b3, wp1, bp1, wp2, bp2):
    part = pl.BlockSpec((_NC, _BLK, _D), lambda i: (0, i, 0))
    fixed_d = pl.BlockSpec((1, _D), lambda i: (0, 0))
    return pl.pallas_call(
        _readout_body,
        grid=(_NPAD // _BLK,),
        in_specs=[
            part, part, part, part,
            pl.BlockSpec((_BLK, 16), lambda i: (i, 0)),
            fixed_d, fixed_d, fixed_d, fixed_d,
            pl.BlockSpec((_D, _H), lambda i: (0, 0)),
            pl.BlockSpec((1, _H), lambda i: (0, 0)),
            pl.BlockSpec((_H, _D), lambda i: (0, 0)),
            fixed_d,
        ],
        out_specs=pl.BlockSpec((8, _D), lambda i: (0, 0)),
        out_shape=jax.ShapeDtypeStruct((8, _D), jnp.float32),
        scratch_shapes=[pltpu.VMEM((_BLK, _D), jnp.float32)],
    )(p0, p1, p2, p3, deg, b0, b1, b2, b3, wp1, bp1, wp2, bp2)


_MESH = plsc.VectorSubcoreMesh(core_axis_name="c", subcore_axis_name="s")


@functools.partial(
    pl.kernel,
    out_type=jax.ShapeDtypeStruct((_NC, _NPAD, 16), jnp.float32),
    mesh=_MESH,
    scratch_types=[
        pltpu.VMEM((_NB,), jnp.int32),
        pltpu.VMEM((_NB, 16), jnp.float32),
        pltpu.VMEM((_ROWS_PER_TILE, 16), jnp.float32),
        pltpu.VMEM_SHARED((_NPAD, 16), jnp.float32),
    ],
)
def _deg_call(idx_hbm, out_hbm, idx_v, ones_v, zbuf_v, acc_sh):
    cid = lax.axis_index("c")
    sid = lax.axis_index("s")
    wid = cid * _NS + sid

    def zfill(i, carry):
        zbuf_v[i, :] = jnp.zeros((16,), jnp.float32)
        return carry

    lax.fori_loop(0, _ROWS_PER_TILE, zfill, 0)
    pltpu.sync_copy(zbuf_v, acc_sh.at[pl.ds(sid * _ROWS_PER_TILE, _ROWS_PER_TILE)])
    plsc.subcore_barrier()

    for h in range(8):
        onehot = (lax.iota(jnp.int32, 16) == h).astype(jnp.float32)

        def ofill(i, carry, onehot=onehot):
            ones_v[i, :] = onehot
            return carry

        lax.fori_loop(0, _NB, ofill, 0)

        def step(j, carry, h=h):
            base = wid * (_EBLK * _NB) + j * _NB
            pltpu.sync_copy(idx_hbm.at[h, pl.ds(base, _NB)], idx_v)
            pltpu.sync_copy(ones_v, acc_sh.at[idx_v], add=True)
            return carry

        lax.fori_loop(0, _EBLK, step, 0)

    plsc.subcore_barrier()
    pltpu.sync_copy(
        acc_sh.at[pl.ds(sid * _ROWS_PER_TILE, _ROWS_PER_TILE)],
        out_hbm.at[cid, pl.ds(sid * _ROWS_PER_TILE, _ROWS_PER_TILE)])


@functools.partial(
    pl.kernel,
    out_type=jax.ShapeDtypeStruct((4, _NC, _NPAD, _D), jnp.float32),
    mesh=_MESH,
    scratch_types=[
        pltpu.VMEM((_NB,), jnp.int32),
        pltpu.VMEM((_NB,), jnp.int32),
        pltpu.VMEM((_NB, _D), jnp.float32),
        pltpu.VMEM((_NB, _D), jnp.float32),
        pltpu.VMEM_SHARED((_NPAD, _D), jnp.float32),
        pltpu.SemaphoreType.DMA,
    ],
)
def _scatter_call(y0, y1, y2, y3, src_hbm, dst_hbm, out_hbm,
                  idx_s, idx_d, rows_v, zbuf_v, acc_sh, sem):
    cid = lax.axis_index("c")
    sid = lax.axis_index("s")
    wid = cid * _NS + sid
    ys = (y0, y1, y2, y3)

    def zfill(i, carry):
        for j in range(_D // 16):
            zbuf_v[i, pl.ds(j * 16, 16)] = jnp.zeros((16,), jnp.float32)
        return carry

    lax.fori_loop(0, _NB, zfill, 0)

    for r in range(4):
        for k in range(_ROWS_PER_TILE // _NB):
            pltpu.sync_copy(
                zbuf_v,
                acc_sh.at[pl.ds(sid * _ROWS_PER_TILE + k * _NB, _NB)])
        plsc.subcore_barrier()

        def step(j, carry, r=r):
            base = wid * (_EBLK * _NB) + j * _NB
            pltpu.sync_copy(src_hbm.at[r, pl.ds(base, _NB)], idx_s)
            pltpu.sync_copy(dst_hbm.at[r, pl.ds(base, _NB)], idx_d)
            pltpu.async_copy(ys[r].at[idx_s], rows_v, sem).wait()
            pltpu.sync_copy(rows_v, acc_sh.at[idx_d], add=True)
            return carry

        lax.fori_loop(0, _EBLK, step, 0)
        plsc.subcore_barrier()
        pltpu.sync_copy(
            acc_sh.at[pl.ds(sid * _ROWS_PER_TILE, _ROWS_PER_TILE)],
            out_hbm.at[r, cid, pl.ds(sid * _ROWS_PER_TILE, _ROWS_PER_TILE)])


def _enc_fold(ep):
    c = 1.0 / math.sqrt(1.0 + 1e-3)
    c1 = ep['g1'] * c
    d1 = ep['b1'] * c1 + ep['be1']
    c2 = ep['g2'] * c
    d2 = ep['b2'] * c2 + ep['be2']
    return (ep['W1'] * c1[None, :], d1.reshape(1, -1),
            ep['W2'] * c2[None, :], d2.reshape(1, -1))


def kernel(x_gene, x_protein, ei_gene_to_gene, ei_gene_to_protein,
           ei_protein_to_gene, ei_protein_to_protein, params):
    p = params

    def pad_e(ei):
        pad = jnp.full((2, _EPAD - _E), _PAD_NODE, jnp.int32)
        return jnp.concatenate([ei.astype(jnp.int32), pad], axis=1)

    eis = [pad_e(e) for e in (ei_gene_to_gene, ei_gene_to_protein,
                              ei_protein_to_gene, ei_protein_to_protein)]
    srcs = jnp.stack([e[0] for e in eis])
    dsts = jnp.stack([e[1] for e in eis])
    hist_idx = jnp.stack([eis[r][e] for r in range(4) for e in range(2)])

    degp = _deg_call(hist_idx)
    deg = degp[0] + degp[1]

    xg = jnp.pad(x_gene, ((0, _NPAD - _N), (0, 0)))
    xp = jnp.pad(x_protein, ((0, _NPAD - _N), (0, 0)))
    fg = _encode(xg, *_enc_fold(p['enc_gene']))
    fp = _encode(xp, *_enc_fold(p['enc_protein']))

    def b2d(b):
        return b.reshape(1, _D)

    y0, y1 = _xw1(fg, p['gnn0_gene_to_gene']['W'], p['gnn0_gene_to_protein']['W'], deg, 0, 2)
    y2, y3 = _xw1(fp, p['gnn0_protein_to_gene']['W'], p['gnn0_protein_to_protein']['W'], deg, 4, 6)
    acc = _scatter_call(y0, y1, y2, y3, srcs, dsts)

    y0, y1 = _xw2(acc[0], acc[2], deg, 1, 5,
                  b2d(p['gnn0_gene_to_gene']['b']), b2d(p['gnn0_protein_to_gene']['b']),
                  p['gnn1_gene_to_gene']['W'], p['gnn1_gene_to_protein']['W'], 0, 2)
    y2, y3 = _xw2(acc[1], acc[3], deg, 3, 7,
                  b2d(p['gnn0_gene_to_protein']['b']), b2d(p['gnn0_protein_to_protein']['b']),
                  p['gnn1_protein_to_gene']['W'], p['gnn1_protein_to_protein']['W'], 4, 6)
    acc = _scatter_call(y0, y1, y2, y3, srcs, dsts)

    pp = p['pred']
    out = _readout(
        acc[0], acc[1], acc[2], acc[3], deg,
        b2d(p['gnn1_gene_to_gene']['b']), b2d(p['gnn1_gene_to_protein']['b']),
        b2d(p['gnn1_protein_to_gene']['b']), b2d(p['gnn1_protein_to_protein']['b']),
        pp['W1'], pp['b1'].reshape(1, _H),
        jnp.pad(pp['W2'], ((0, 0), (0, _D - 1))),
        jnp.broadcast_to(pp['b2'].reshape(1, 1), (1, _D)))
    return out[0, 0:1]


# speedup vs baseline: 4.8064x; 4.8064x over previous
import functools
import math

import jax
import jax.numpy as jnp
from jax import lax
from jax.experimental import pallas as pl
from jax.experimental.pallas import tpu as pltpu
from jax.experimental.pallas import tpu_sc as plsc

_N = 10000
_NPAD = 10240
_E = 160000
_EPAD = 163840
_NB = 128
_EBLK = 40
_PAD_NODE = 10200
_D = 128
_H = 256
_BLK = 256
_NC = 2
_NS = 16
_ROWS_PER_TILE = _NPAD // _NS


def _rs(x):
    return lax.rsqrt(jnp.maximum(x, 1.0))


def _enc_body(x_ref, w1_ref, c1_ref, d1_ref, w2_ref, c2_ref, d2_ref, o_ref):
    h = jnp.dot(x_ref[...], w1_ref[...], preferred_element_type=jnp.float32)
    h = jnp.maximum(h * c1_ref[...] + d1_ref[...], 0.0)
    o = jnp.dot(h, w2_ref[...], preferred_element_type=jnp.float32)
    o_ref[...] = o * c2_ref[...] + d2_ref[...]


def _encode(x, w1, c1, d1, w2, c2, d2):
    din = x.shape[1]
    return pl.pallas_call(
        _enc_body,
        grid=(_NPAD // _BLK,),
        in_specs=[
            pl.BlockSpec((_BLK, din), lambda i: (i, 0)),
            pl.BlockSpec((din, _H), lambda i: (0, 0)),
            pl.BlockSpec((1, _H), lambda i: (0, 0)),
            pl.BlockSpec((1, _H), lambda i: (0, 0)),
            pl.BlockSpec((_H, _D), lambda i: (0, 0)),
            pl.BlockSpec((1, _D), lambda i: (0, 0)),
            pl.BlockSpec((1, _D), lambda i: (0, 0)),
        ],
        out_specs=pl.BlockSpec((_BLK, _D), lambda i: (i, 0)),
        out_shape=jax.ShapeDtypeStruct((_NPAD, _D), jnp.float32),
    )(x, w1, c1, d1, w2, c2, d2)


def _xw1_body(ha, hb, f_ref, wa_ref, wb_ref, deg_ref, ya_ref, yb_ref):
    f = f_ref[...]
    sa = _rs(deg_ref[:, ha:ha + 1])
    sb = _rs(deg_ref[:, hb:hb + 1])
    ya_ref[...] = jnp.dot(f, wa_ref[...], preferred_element_type=jnp.float32) * sa
    yb_ref[...] = jnp.dot(f, wb_ref[...], preferred_element_type=jnp.float32) * sb


def _xw1(f, wa, wb, deg, ha, hb):
    return pl.pallas_call(
        functools.partial(_xw1_body, ha, hb),
        grid=(_NPAD // _BLK,),
        in_specs=[
            pl.BlockSpec((_BLK, _D), lambda i: (i, 0)),
            pl.BlockSpec((_D, _D), lambda i: (0, 0)),
            pl.BlockSpec((_D, _D), lambda i: (0, 0)),
            pl.BlockSpec((_BLK, 8), lambda i: (i, 0)),
        ],
        out_specs=[
            pl.BlockSpec((_BLK, _D), lambda i: (i, 0)),
            pl.BlockSpec((_BLK, _D), lambda i: (i, 0)),
        ],
        out_shape=[
            jax.ShapeDtypeStruct((_NPAD, _D), jnp.float32),
            jax.ShapeDtypeStruct((_NPAD, _D), jnp.float32),
        ],
    )(f, wa, wb, deg)


def _xw2_body(hdA, hdB, ha, hb, pA_ref, pB_ref, deg_ref, bA_ref, bB_ref,
              wa_ref, wb_ref, ya_ref, yb_ref):
    gA = (pA_ref[0] + pA_ref[1]) * _rs(deg_ref[:, hdA:hdA + 1]) + bA_ref[...]
    gB = (pB_ref[0] + pB_ref[1]) * _rs(deg_ref[:, hdB:hdB + 1]) + bB_ref[...]
    f = jnp.maximum(gA + gB, 0.0)
    sa = _rs(deg_ref[:, ha:ha + 1])
    sb = _rs(deg_ref[:, hb:hb + 1])
    ya_ref[...] = jnp.dot(f, wa_ref[...], preferred_element_type=jnp.float32) * sa
    yb_ref[...] = jnp.dot(f, wb_ref[...], preferred_element_type=jnp.float32) * sb


def _xw2(pA, pB, deg, hdA, hdB, bA, bB, wa, wb, ha, hb):
    return pl.pallas_call(
        functools.partial(_xw2_body, hdA, hdB, ha, hb),
        grid=(_NPAD // _BLK,),
        in_specs=[
            pl.BlockSpec((_NC, _BLK, _D), lambda i: (0, i, 0)),
            pl.BlockSpec((_NC, _BLK, _D), lambda i: (0, i, 0)),
            pl.BlockSpec((_BLK, 8), lambda i: (i, 0)),
            pl.BlockSpec((1, _D), lambda i: (0, 0)),
            pl.BlockSpec((1, _D), lambda i: (0, 0)),
            pl.BlockSpec((_D, _D), lambda i: (0, 0)),
            pl.BlockSpec((_D, _D), lambda i: (0, 0)),
        ],
        out_specs=[
            pl.BlockSpec((_BLK, _D), lambda i: (i, 0)),
            pl.BlockSpec((_BLK, _D), lambda i: (i, 0)),
        ],
        out_shape=[
            jax.ShapeDtypeStruct((_NPAD, _D), jnp.float32),
            jax.ShapeDtypeStruct((_NPAD, _D), jnp.float32),
        ],
    )(pA, pB, deg, bA, bB, wa, wb)


def _readout_body(p0_ref, p1_ref, p2_ref, p3_ref, deg_ref,
                  b0_ref, b1_ref, b2_ref, b3_ref,
                  wp1_ref, bp1_ref, wp2_ref, bp2_ref, o_ref, s_ref):
    i = pl.program_id(0)
    fg = jnp.maximum(
        (p0_ref[0] + p0_ref[1]) * _rs(deg_ref[:, 1:2]) + b0_ref[...]
        + (p2_ref[0] + p2_ref[1]) * _rs(deg_ref[:, 5:6]) + b2_ref[...], 0.0)
    fp = jnp.maximum(
        (p1_ref[0] + p1_ref[1]) * _rs(deg_ref[:, 3:4]) + b1_ref[...]
        + (p3_ref[0] + p3_ref[1]) * _rs(deg_ref[:, 7:8]) + b3_ref[...], 0.0)
    rows = i * _BLK + lax.broadcasted_iota(jnp.int32, (_BLK, 1), 0)
    contrib = jnp.where(rows < _N, fg + fp, 0.0)

    @pl.when(i == 0)
    def _():
        s_ref[...] = jnp.zeros_like(s_ref)

    s_ref[...] += contrib

    @pl.when(i == _NPAD // _BLK - 1)
    def _():
        g = jnp.sum(s_ref[...], axis=0, keepdims=True) * (1.0 / (2 * _N))
        h = jnp.maximum(
            jnp.dot(g, wp1_ref[...], preferred_element_type=jnp.float32)
            + bp1_ref[...], 0.0)
        out = jnp.dot(h, wp2_ref[...], preferred_element_type=jnp.float32)
        o_ref[...] = jnp.broadcast_to(out + bp2_ref[...], (8, _D))


def _readout(p0, p1, p2, p3, deg, b0, b1, b2, b3, wp1, bp1, wp2, bp2):
    part = pl.BlockSpec((_NC, _BLK, _D), lambda i: (0, i, 0))
    fixed_d = pl.BlockSpec((1, _D), lambda i: (0, 0))
    return pl.pallas_call(
        _readout_body,
        grid=(_NPAD // _BLK,),
        in_specs=[
            part, part, part, part,
            pl.BlockSpec((_BLK, 8), lambda i: (i, 0)),
            fixed_d, fixed_d, fixed_d, fixed_d,
            pl.BlockSpec((_D, _H), lambda i: (0, 0)),
            pl.BlockSpec((1, _H), lambda i: (0, 0)),
            pl.BlockSpec((_H, _D), lambda i: (0, 0)),
            fixed_d,
        ],
        out_specs=pl.BlockSpec((8, _D), lambda i: (0, 0)),
        out_shape=jax.ShapeDtypeStruct((8, _D), jnp.float32),
        scratch_shapes=[pltpu.VMEM((_BLK, _D), jnp.float32)],
    )(p0, p1, p2, p3, deg, b0, b1, b2, b3, wp1, bp1, wp2, bp2)


@functools.cache
def _mesh():
    return plsc.VectorSubcoreMesh(core_axis_name="c", subcore_axis_name="s")


def _deg_call(idx_hbm_arr):
    ones = jnp.ones((_NB,), jnp.float32)
    z1 = jnp.zeros((_ROWS_PER_TILE,), jnp.float32)
    return _build_deg()(idx_hbm_arr, ones, z1)


@functools.cache
def _build_deg():
    return functools.partial(
        pl.kernel,
        out_type=jax.ShapeDtypeStruct((_NC, 8, _NPAD), jnp.float32),
        mesh=_mesh(),
        scratch_types=[
            pltpu.VMEM((_NB,), jnp.int32),
            pltpu.VMEM((_NB,), jnp.float32),
            pltpu.VMEM((_ROWS_PER_TILE,), jnp.float32),
        ] + [pltpu.VMEM_SHARED((_NPAD,), jnp.float32) for _ in range(8)],
    )(_deg_body)


def _deg_body(idx_hbm, ones_hbm, z_hbm, out_hbm, idx_v, ones_v, zbuf_v, *accs):
    cid = lax.axis_index("c")
    sid = lax.axis_index("s")
    wid = cid * _NS + sid

    pltpu.sync_copy(ones_hbm, ones_v)
    pltpu.sync_copy(z_hbm, zbuf_v)
    sl = pl.ds(sid * _ROWS_PER_TILE, _ROWS_PER_TILE)
    for h in range(8):
        pltpu.sync_copy(zbuf_v, accs[h].at[sl])
    plsc.subcore_barrier()

    for h in range(8):
        def step(j, carry, h=h):
            base = wid * (_EBLK * _NB) + j * _NB
            pltpu.sync_copy(idx_hbm.at[h, pl.ds(base, _NB)], idx_v)
            pltpu.sync_copy(ones_v, accs[h].at[idx_v], add=True)
            return carry

        lax.fori_loop(0, _EBLK, step, 0)

    plsc.subcore_barrier()
    for h in range(8):
        pltpu.sync_copy(accs[h].at[sl], out_hbm.at[cid, h, sl])


def _scatter_call(y0, y1, y2, y3, srcs, dsts):
    zD = jnp.zeros((_NB, _D), jnp.float32)
    return _build_scatter()(y0, y1, y2, y3, srcs, dsts, zD)


@functools.cache
def _build_scatter():
    return functools.partial(
        pl.kernel,
        out_type=jax.ShapeDtypeStruct((4, _NC, _NPAD, _D), jnp.float32),
        mesh=_mesh(),
        scratch_types=[
            pltpu.VMEM((_NB,), jnp.int32),
            pltpu.VMEM((_NB,), jnp.int32),
            pltpu.VMEM((_NB, _D), jnp.float32),
            pltpu.VMEM((_NB, _D), jnp.float32),
            pltpu.VMEM_SHARED((_NPAD, _D), jnp.float32),
            pltpu.SemaphoreType.DMA,
        ],
    )(_scatter_body)


def _scatter_body(y0, y1, y2, y3, src_hbm, dst_hbm, zD_hbm, out_hbm,
                  idx_s, idx_d, rows_v, zbuf_v, acc_sh, sem):
    cid = lax.axis_index("c")
    sid = lax.axis_index("s")
    wid = cid * _NS + sid
    ys = (y0, y1, y2, y3)

    pltpu.sync_copy(zD_hbm, zbuf_v)

    for r in range(4):
        for k in range(_ROWS_PER_TILE // _NB):
            pltpu.sync_copy(
                zbuf_v,
                acc_sh.at[pl.ds(sid * _ROWS_PER_TILE + k * _NB, _NB)])
        plsc.subcore_barrier()

        def step(j, carry, r=r):
            base = wid * (_EBLK * _NB) + j * _NB
            pltpu.sync_copy(src_hbm.at[r, pl.ds(base, _NB)], idx_s)
            pltpu.sync_copy(dst_hbm.at[r, pl.ds(base, _NB)], idx_d)
            pltpu.async_copy(ys[r].at[idx_s], rows_v, sem).wait()
            pltpu.sync_copy(rows_v, acc_sh.at[idx_d], add=True)
            return carry

        lax.fori_loop(0, _EBLK, step, 0)
        plsc.subcore_barrier()
        pltpu.sync_copy(
            acc_sh.at[pl.ds(sid * _ROWS_PER_TILE, _ROWS_PER_TILE)],
            out_hbm.at[r, cid, pl.ds(sid * _ROWS_PER_TILE, _ROWS_PER_TILE)])


def _enc_fold(ep):
    c = 1.0 / math.sqrt(1.0 + 1e-3)
    c1 = ep['g1'] * c
    d1 = ep['b1'] * c1 + ep['be1']
    c2 = ep['g2'] * c
    d2 = ep['b2'] * c2 + ep['be2']
    return (ep['W1'], c1.reshape(1, -1), d1.reshape(1, -1),
            ep['W2'], c2.reshape(1, -1), d2.reshape(1, -1))


def kernel(x_gene, x_protein, ei_gene_to_gene, ei_gene_to_protein,
           ei_protein_to_gene, ei_protein_to_protein, params):
    p = params

    def pad_e(ei):
        pad = jnp.full((2, _EPAD - _E), _PAD_NODE, jnp.int32)
        return jnp.concatenate([ei.astype(jnp.int32), pad], axis=1)

    eis = [pad_e(e) for e in (ei_gene_to_gene, ei_gene_to_protein,
                              ei_protein_to_gene, ei_protein_to_protein)]
    srcs = jnp.stack([e[0] for e in eis])
    dsts = jnp.stack([e[1] for e in eis])
    hist_idx = jnp.stack([eis[r][e] for r in range(4) for e in range(2)])

    degp = _deg_call(hist_idx)
    deg = (degp[0] + degp[1]).T

    xg = jnp.pad(x_gene, ((0, _NPAD - _N), (0, 0)))
    xp = jnp.pad(x_protein, ((0, _NPAD - _N), (0, 0)))
    fg = _encode(xg, *_enc_fold(p['enc_gene']))
    fp = _encode(xp, *_enc_fold(p['enc_protein']))

    def b2d(b):
        return b.reshape(1, _D)

    y0, y1 = _xw1(fg, p['gnn0_gene_to_gene']['W'], p['gnn0_gene_to_protein']['W'], deg, 0, 2)
    y2, y3 = _xw1(fp, p['gnn0_protein_to_gene']['W'], p['gnn0_protein_to_protein']['W'], deg, 4, 6)
    acc = _scatter_call(y0, y1, y2, y3, srcs, dsts)

    y0, y1 = _xw2(acc[0], acc[2], deg, 1, 5,
                  b2d(p['gnn0_gene_to_gene']['b']), b2d(p['gnn0_protein_to_gene']['b']),
                  p['gnn1_gene_to_gene']['W'], p['gnn1_gene_to_protein']['W'], 0, 2)
    y2, y3 = _xw2(acc[1], acc[3], deg, 3, 7,
                  b2d(p['gnn0_gene_to_protein']['b']), b2d(p['gnn0_protein_to_protein']['b']),
                  p['gnn1_protein_to_gene']['W'], p['gnn1_protein_to_protein']['W'], 4, 6)
    acc = _scatter_call(y0, y1, y2, y3, srcs, dsts)

    pp = p['pred']
    out = _readout(
        acc[0], acc[1], acc[2], acc[3], deg,
        b2d(p['gnn1_gene_to_gene']['b']), b2d(p['gnn1_gene_to_protein']['b']),
        b2d(p['gnn1_protein_to_gene']['b']), b2d(p['gnn1_protein_to_protein']['b']),
        pp['W1'], pp['b1'].reshape(1, _H),
        jnp.pad(pp['W2'], ((0, 0), (0, _D - 1))),
        jnp.broadcast_to(pp['b2'].reshape(1, 1), (1, _D)))
    return out[0, 0:1]


# speedup vs baseline: 5.9720x; 1.2425x over previous
import functools
import math

import jax
import jax.numpy as jnp
from jax import lax
from jax.experimental import pallas as pl
from jax.experimental.pallas import tpu as pltpu
from jax.experimental.pallas import tpu_sc as plsc

_N = 10000
_NPAD = 10240
_E = 160000
_EPAD = 163840
_NB = 128
_EBLK = 40
_PAD_NODE = 10050
_D = 128
_H = 256
_BLK = 256
_NC = 2
_NS = 16
_ROWS_PER_TILE = _NPAD // _NS
_APAD = 10112
_ART = _APAD // _NS
_ZCHUNKS = [(0, 128), (128, 128), (256, 128), (384, 128), (512, 120)]


def _rs(x):
    return lax.rsqrt(jnp.maximum(x, 1.0))


def _enc_body(x_ref, w1_ref, c1_ref, d1_ref, w2_ref, c2_ref, d2_ref, o_ref):
    h = jnp.dot(x_ref[...], w1_ref[...], preferred_element_type=jnp.float32)
    h = jnp.maximum(h * c1_ref[...] + d1_ref[...], 0.0)
    o = jnp.dot(h, w2_ref[...], preferred_element_type=jnp.float32)
    o_ref[...] = o * c2_ref[...] + d2_ref[...]


def _encode(x, w1, c1, d1, w2, c2, d2):
    din = x.shape[1]
    return pl.pallas_call(
        _enc_body,
        grid=(_NPAD // _BLK,),
        in_specs=[
            pl.BlockSpec((_BLK, din), lambda i: (i, 0)),
            pl.BlockSpec((din, _H), lambda i: (0, 0)),
            pl.BlockSpec((1, _H), lambda i: (0, 0)),
            pl.BlockSpec((1, _H), lambda i: (0, 0)),
            pl.BlockSpec((_H, _D), lambda i: (0, 0)),
            pl.BlockSpec((1, _D), lambda i: (0, 0)),
            pl.BlockSpec((1, _D), lambda i: (0, 0)),
        ],
        out_specs=pl.BlockSpec((_BLK, _D), lambda i: (i, 0)),
        out_shape=jax.ShapeDtypeStruct((_NPAD, _D), jnp.float32),
    )(x, w1, c1, d1, w2, c2, d2)


def _xw1_body(ha, hb, f_ref, wa_ref, wb_ref, deg_ref, ya_ref, yb_ref):
    f = f_ref[...]
    sa = _rs(deg_ref[:, ha:ha + 1])
    sb = _rs(deg_ref[:, hb:hb + 1])
    ya_ref[...] = jnp.dot(f, wa_ref[...], preferred_element_type=jnp.float32) * sa
    yb_ref[...] = jnp.dot(f, wb_ref[...], preferred_element_type=jnp.float32) * sb


def _xw1(f, wa, wb, deg, ha, hb):
    return pl.pallas_call(
        functools.partial(_xw1_body, ha, hb),
        grid=(_NPAD // _BLK,),
        in_specs=[
            pl.BlockSpec((_BLK, _D), lambda i: (i, 0)),
            pl.BlockSpec((_D, _D), lambda i: (0, 0)),
            pl.BlockSpec((_D, _D), lambda i: (0, 0)),
            pl.BlockSpec((_BLK, 8), lambda i: (i, 0)),
        ],
        out_specs=[
            pl.BlockSpec((_BLK, _D), lambda i: (i, 0)),
            pl.BlockSpec((_BLK, _D), lambda i: (i, 0)),
        ],
        out_shape=[
            jax.ShapeDtypeStruct((_NPAD, _D), jnp.float32),
            jax.ShapeDtypeStruct((_NPAD, _D), jnp.float32),
        ],
    )(f, wa, wb, deg)


def _xw2_body(hdA, hdB, ha, hb, pA_ref, pB_ref, deg_ref, bA_ref, bB_ref,
              wa_ref, wb_ref, ya_ref, yb_ref):
    gA = (pA_ref[0] + pA_ref[1]) * _rs(deg_ref[:, hdA:hdA + 1]) + bA_ref[...]
    gB = (pB_ref[0] + pB_ref[1]) * _rs(deg_ref[:, hdB:hdB + 1]) + bB_ref[...]
    f = jnp.maximum(gA + gB, 0.0)
    sa = _rs(deg_ref[:, ha:ha + 1])
    sb = _rs(deg_ref[:, hb:hb + 1])
    ya_ref[...] = jnp.dot(f, wa_ref[...], preferred_element_type=jnp.float32) * sa
    yb_ref[...] = jnp.dot(f, wb_ref[...], preferred_element_type=jnp.float32) * sb


def _xw2(pA, pB, deg, hdA, hdB, bA, bB, wa, wb, ha, hb):
    return pl.pallas_call(
        functools.partial(_xw2_body, hdA, hdB, ha, hb),
        grid=(_NPAD // _BLK,),
        in_specs=[
            pl.BlockSpec((_NC, _BLK, _D), lambda i: (0, i, 0)),
            pl.BlockSpec((_NC, _BLK, _D), lambda i: (0, i, 0)),
            pl.BlockSpec((_BLK, 8), lambda i: (i, 0)),
            pl.BlockSpec((1, _D), lambda i: (0, 0)),
            pl.BlockSpec((1, _D), lambda i: (0, 0)),
            pl.BlockSpec((_D, _D), lambda i: (0, 0)),
            pl.BlockSpec((_D, _D), lambda i: (0, 0)),
        ],
        out_specs=[
            pl.BlockSpec((_BLK, _D), lambda i: (i, 0)),
            pl.BlockSpec((_BLK, _D), lambda i: (i, 0)),
        ],
        out_shape=[
            jax.ShapeDtypeStruct((_NPAD, _D), jnp.float32),
            jax.ShapeDtypeStruct((_NPAD, _D), jnp.float32),
        ],
    )(pA, pB, deg, bA, bB, wa, wb)


def _readout_body(p0_ref, p1_ref, p2_ref, p3_ref, deg_ref,
                  b0_ref, b1_ref, b2_ref, b3_ref,
                  wp1_ref, bp1_ref, wp2_ref, bp2_ref, o_ref, s_ref):
    i = pl.program_id(0)
    fg = jnp.maximum(
        (p0_ref[0] + p0_ref[1]) * _rs(deg_ref[:, 1:2]) + b0_ref[...]
        + (p2_ref[0] + p2_ref[1]) * _rs(deg_ref[:, 5:6]) + b2_ref[...], 0.0)
    fp = jnp.maximum(
        (p1_ref[0] + p1_ref[1]) * _rs(deg_ref[:, 3:4]) + b1_ref[...]
        + (p3_ref[0] + p3_ref[1]) * _rs(deg_ref[:, 7:8]) + b3_ref[...], 0.0)
    rows = i * _BLK + lax.broadcasted_iota(jnp.int32, (_BLK, 1), 0)
    contrib = jnp.where(rows < _N, fg + fp, 0.0)

    @pl.when(i == 0)
    def _():
        s_ref[...] = jnp.zeros_like(s_ref)

    s_ref[...] += contrib

    @pl.when(i == _NPAD // _BLK - 1)
    def _():
        g = jnp.sum(s_ref[...], axis=0, keepdims=True) * (1.0 / (2 * _N))
        h = jnp.maximum(
            jnp.dot(g, wp1_ref[...], preferred_element_type=jnp.float32)
            + bp1_ref[...], 0.0)
        out = jnp.dot(h, wp2_ref[...], preferred_element_type=jnp.float32)
        o_ref[...] = jnp.broadcast_to(out + bp2_ref[...], (8, _D))


def _readout(p0, p1, p2, p3, deg, b0, b1, b2, b3, wp1, bp1, wp2, bp2):
    part = pl.BlockSpec((_NC, _BLK, _D), lambda i: (0, i, 0))
    fixed_d = pl.BlockSpec((1, _D), lambda i: (0, 0))
    return pl.pallas_call(
        _readout_body,
        grid=(_NPAD // _BLK,),
        in_specs=[
            part, part, part, part,
            pl.BlockSpec((_BLK, 8), lambda i: (i, 0)),
            fixed_d, fixed_d, fixed_d, fixed_d,
            pl.BlockSpec((_D, _H), lambda i: (0, 0)),
            pl.BlockSpec((1, _H), lambda i: (0, 0)),
            pl.BlockSpec((_H, _D), lambda i: (0, 0)),
            fixed_d,
        ],
        out_specs=pl.BlockSpec((8, _D), lambda i: (0, 0)),
        out_shape=jax.ShapeDtypeStruct((8, _D), jnp.float32),
        scratch_shapes=[pltpu.VMEM((_BLK, _D), jnp.float32)],
    )(p0, p1, p2, p3, deg, b0, b1, b2, b3, wp1, bp1, wp2, bp2)


@functools.cache
def _mesh():
    return plsc.VectorSubcoreMesh(core_axis_name="c", subcore_axis_name="s")


def _deg_call(idx_hbm_arr):
    ones = jnp.ones((_NB,), jnp.float32)
    z1 = jnp.zeros((_ROWS_PER_TILE,), jnp.float32)
    return _build_deg()(idx_hbm_arr, ones, z1)


@functools.cache
def _build_deg():
    return functools.partial(
        pl.kernel,
        out_type=jax.ShapeDtypeStruct((_NC, 8, _NPAD), jnp.float32),
        mesh=_mesh(),
        scratch_types=[
            pltpu.VMEM((_NB,), jnp.int32),
            pltpu.VMEM((_NB,), jnp.float32),
            pltpu.VMEM((_ROWS_PER_TILE,), jnp.float32),
        ] + [pltpu.VMEM_SHARED((_NPAD,), jnp.float32) for _ in range(8)],
    )(_deg_body)


def _deg_body(idx_hbm, ones_hbm, z_hbm, out_hbm, idx_v, ones_v, zbuf_v, *accs):
    cid = lax.axis_index("c")
    sid = lax.axis_index("s")
    wid = cid * _NS + sid

    pltpu.sync_copy(ones_hbm, ones_v)
    pltpu.sync_copy(z_hbm, zbuf_v)
    sl = pl.ds(sid * _ROWS_PER_TILE, _ROWS_PER_TILE)
    for h in range(8):
        pltpu.sync_copy(zbuf_v, accs[h].at[sl])
    plsc.subcore_barrier()

    for h in range(8):
        def step(j, carry, h=h):
            base = wid * (_EBLK * _NB) + j * _NB
            pltpu.sync_copy(idx_hbm.at[h, pl.ds(base, _NB)], idx_v)
            pltpu.sync_copy(ones_v, accs[h].at[idx_v], add=True)
            return carry

        lax.fori_loop(0, _EBLK, step, 0)

    plsc.subcore_barrier()
    for h in range(8):
        pltpu.sync_copy(accs[h].at[sl], out_hbm.at[cid, h, sl])


def _scatter_call(y0, y1, y2, y3, srcs, dsts):
    zD = jnp.zeros((_NB, _D), jnp.float32)
    return _build_scatter()(y0, y1, y2, y3, srcs, dsts, zD)


@functools.cache
def _build_scatter():
    return functools.partial(
        pl.kernel,
        out_type=jax.ShapeDtypeStruct((4, _NC, _NPAD, _D), jnp.float32),
        mesh=_mesh(),
        scratch_types=[
            pltpu.VMEM((_NB,), jnp.int32),
            pltpu.VMEM((_NB,), jnp.int32),
            pltpu.VMEM((_NB,), jnp.int32),
            pltpu.VMEM((_NB,), jnp.int32),
            pltpu.VMEM((_NB, _D), jnp.float32),
            pltpu.VMEM((_NB, _D), jnp.float32),
            pltpu.VMEM((_NB, _D), jnp.float32),
            pltpu.VMEM_SHARED((_APAD, _D), jnp.float32),
            pltpu.SemaphoreType.DMA,
            pltpu.SemaphoreType.DMA,
        ],
    )(_scatter_body)


def _scatter_body(y0, y1, y2, y3, src_hbm, dst_hbm, zD_hbm, out_hbm,
                  s0, s1, d0, d1, rows0, rows1, zbuf_v, acc_sh, sem0, sem1):
    cid = lax.axis_index("c")
    sid = lax.axis_index("s")
    wid = cid * _NS + sid
    ys = (y0, y1, y2, y3)

    pltpu.sync_copy(zD_hbm, zbuf_v)

    for r in range(4):
        for k, (off, sz) in enumerate(_ZCHUNKS):
            pltpu.sync_copy(
                zbuf_v.at[pl.ds(0, sz)],
                acc_sh.at[pl.ds(sid * _ART + off, sz)])
        plsc.subcore_barrier()

        tb = wid * (_EBLK * _NB)
        pltpu.sync_copy(src_hbm.at[r, pl.ds(tb, _NB)], s0)
        pltpu.async_copy(ys[r].at[s0], rows0, sem0)

        def pair(jj, carry, r=r, tb=tb):
            b0 = tb + (2 * jj) * _NB
            b1 = b0 + _NB
            pltpu.sync_copy(src_hbm.at[r, pl.ds(b1, _NB)], s1)
            pltpu.async_copy(ys[r].at[s1], rows1, sem1)
            pltpu.make_async_copy(ys[r].at[s0], rows0, sem0).wait()
            pltpu.sync_copy(dst_hbm.at[r, pl.ds(b0, _NB)], d0)
            pltpu.sync_copy(rows0, acc_sh.at[d0], add=True)

            @pl.when(jj < _EBLK // 2 - 1)
            def _():
                b2 = b0 + 2 * _NB
                pltpu.sync_copy(src_hbm.at[r, pl.ds(b2, _NB)], s0)
                pltpu.async_copy(ys[r].at[s0], rows0, sem0)

            pltpu.make_async_copy(ys[r].at[s1], rows1, sem1).wait()
            pltpu.sync_copy(dst_hbm.at[r, pl.ds(b1, _NB)], d1)
            pltpu.sync_copy(rows1, acc_sh.at[d1], add=True)
            return carry

        lax.fori_loop(0, _EBLK // 2, pair, 0)
        plsc.subcore_barrier()
        pltpu.sync_copy(
            acc_sh.at[pl.ds(sid * _ART, _ART)],
            out_hbm.at[r, cid, pl.ds(sid * _ART, _ART)])


def _enc_fold(ep):
    c = 1.0 / math.sqrt(1.0 + 1e-3)
    c1 = ep['g1'] * c
    d1 = ep['b1'] * c1 + ep['be1']
    c2 = ep['g2'] * c
    d2 = ep['b2'] * c2 + ep['be2']
    return (ep['W1'], c1.reshape(1, -1), d1.reshape(1, -1),
            ep['W2'], c2.reshape(1, -1), d2.reshape(1, -1))


def kernel(x_gene, x_protein, ei_gene_to_gene, ei_gene_to_protein,
           ei_protein_to_gene, ei_protein_to_protein, params):
    p = params

    def pad_e(ei):
        pad = jnp.full((2, _EPAD - _E), _PAD_NODE, jnp.int32)
        return jnp.concatenate([ei.astype(jnp.int32), pad], axis=1)

    eis = [pad_e(e) for e in (ei_gene_to_gene, ei_gene_to_protein,
                              ei_protein_to_gene, ei_protein_to_protein)]
    srcs = jnp.stack([e[0] for e in eis])
    dsts = jnp.stack([e[1] for e in eis])
    hist_idx = jnp.stack([eis[r][e] for r in range(4) for e in range(2)])

    degp = _deg_call(hist_idx)
    deg = (degp[0] + degp[1]).T

    xg = jnp.pad(x_gene, ((0, _NPAD - _N), (0, 0)))
    xp = jnp.pad(x_protein, ((0, _NPAD - _N), (0, 0)))
    fg = _encode(xg, *_enc_fold(p['enc_gene']))
    fp = _encode(xp, *_enc_fold(p['enc_protein']))

    def b2d(b):
        return b.reshape(1, _D)

    y0, y1 = _xw1(fg, p['gnn0_gene_to_gene']['W'], p['gnn0_gene_to_protein']['W'], deg, 0, 2)
    y2, y3 = _xw1(fp, p['gnn0_protein_to_gene']['W'], p['gnn0_protein_to_protein']['W'], deg, 4, 6)
    acc = _scatter_call(y0, y1, y2, y3, srcs, dsts)

    y0, y1 = _xw2(acc[0], acc[2], deg, 1, 5,
                  b2d(p['gnn0_gene_to_gene']['b']), b2d(p['gnn0_protein_to_gene']['b']),
                  p['gnn1_gene_to_gene']['W'], p['gnn1_gene_to_protein']['W'], 0, 2)
    y2, y3 = _xw2(acc[1], acc[3], deg, 3, 7,
                  b2d(p['gnn0_gene_to_protein']['b']), b2d(p['gnn0_protein_to_protein']['b']),
                  p['gnn1_protein_to_gene']['W'], p['gnn1_protein_to_protein']['W'], 4, 6)
    acc = _scatter_call(y0, y1, y2, y3, srcs, dsts)

    pp = p['pred']
    out = _readout(
        acc[0], acc[1], acc[2], acc[3], deg,
        b2d(p['gnn1_gene_to_gene']['b']), b2d(p['gnn1_gene_to_protein']['b']),
        b2d(p['gnn1_protein_to_gene']['b']), b2d(p['gnn1_protein_to_protein']['b']),
        pp['W1'], pp['b1'].reshape(1, _H),
        jnp.pad(pp['W2'], ((0, 0), (0, _D - 1))),
        jnp.broadcast_to(pp['b2'].reshape(1, 1), (1, _D)))
    return out[0, 0:1]


# speedup vs baseline: 12.4017x; 2.0767x over previous
import functools
import math

import jax
import jax.numpy as jnp
from jax import lax
from jax.experimental import pallas as pl
from jax.experimental.pallas import tpu as pltpu
from jax.experimental.pallas import tpu_sc as plsc

_N = 10000
_NPAD = 10240
_E = 160000
_EPAD = 163840
_NB = 128
_EBLK = 40
_PAD_NODE = 10050
_D = 128
_H = 256
_BLK = 256
_NC = 2
_NS = 16
_ROWS_PER_TILE = _NPAD // _NS
_APAD = 10112
_ART = _APAD // _NS
_ZCHUNKS = [(0, 128), (128, 128), (256, 128), (384, 128), (512, 120)]


def _rs(x):
    return lax.rsqrt(jnp.maximum(x, 1.0))


def _enc_body(x_ref, w1_ref, c1_ref, d1_ref, w2_ref, c2_ref, d2_ref, o_ref):
    h = jnp.dot(x_ref[...], w1_ref[...], preferred_element_type=jnp.float32)
    h = jnp.maximum(h * c1_ref[...] + d1_ref[...], 0.0)
    o = jnp.dot(h, w2_ref[...], preferred_element_type=jnp.float32)
    o_ref[...] = o * c2_ref[...] + d2_ref[...]


def _encode(x, w1, c1, d1, w2, c2, d2):
    din = x.shape[1]
    return pl.pallas_call(
        _enc_body,
        grid=(_NPAD // _BLK,),
        in_specs=[
            pl.BlockSpec((_BLK, din), lambda i: (i, 0)),
            pl.BlockSpec((din, _H), lambda i: (0, 0)),
            pl.BlockSpec((1, _H), lambda i: (0, 0)),
            pl.BlockSpec((1, _H), lambda i: (0, 0)),
            pl.BlockSpec((_H, _D), lambda i: (0, 0)),
            pl.BlockSpec((1, _D), lambda i: (0, 0)),
            pl.BlockSpec((1, _D), lambda i: (0, 0)),
        ],
        out_specs=pl.BlockSpec((_BLK, _D), lambda i: (i, 0)),
        out_shape=jax.ShapeDtypeStruct((_NPAD, _D), jnp.float32),
    )(x, w1, c1, d1, w2, c2, d2)


def _xw1_body(ha, hb, f_ref, wa_ref, wb_ref, deg_ref, ya_ref, yb_ref):
    f = f_ref[...]
    sa = _rs(deg_ref[:, ha:ha + 1])
    sb = _rs(deg_ref[:, hb:hb + 1])
    ya_ref[...] = jnp.dot(f, wa_ref[...], preferred_element_type=jnp.float32) * sa
    yb_ref[...] = jnp.dot(f, wb_ref[...], preferred_element_type=jnp.float32) * sb


def _xw1(f, wa, wb, deg, ha, hb):
    return pl.pallas_call(
        functools.partial(_xw1_body, ha, hb),
        grid=(_NPAD // _BLK,),
        in_specs=[
            pl.BlockSpec((_BLK, _D), lambda i: (i, 0)),
            pl.BlockSpec((_D, _D), lambda i: (0, 0)),
            pl.BlockSpec((_D, _D), lambda i: (0, 0)),
            pl.BlockSpec((_BLK, 8), lambda i: (i, 0)),
        ],
        out_specs=[
            pl.BlockSpec((_BLK, _D), lambda i: (i, 0)),
            pl.BlockSpec((_BLK, _D), lambda i: (i, 0)),
        ],
        out_shape=[
            jax.ShapeDtypeStruct((_NPAD, _D), jnp.float32),
            jax.ShapeDtypeStruct((_NPAD, _D), jnp.float32),
        ],
    )(f, wa, wb, deg)


def _xw2_body(hdA, hdB, ha, hb, pA_ref, pB_ref, deg_ref, bA_ref, bB_ref,
              wa_ref, wb_ref, ya_ref, yb_ref):
    gA = (pA_ref[0] + pA_ref[1]) * _rs(deg_ref[:, hdA:hdA + 1]) + bA_ref[...]
    gB = (pB_ref[0] + pB_ref[1]) * _rs(deg_ref[:, hdB:hdB + 1]) + bB_ref[...]
    f = jnp.maximum(gA + gB, 0.0)
    sa = _rs(deg_ref[:, ha:ha + 1])
    sb = _rs(deg_ref[:, hb:hb + 1])
    ya_ref[...] = jnp.dot(f, wa_ref[...], preferred_element_type=jnp.float32) * sa
    yb_ref[...] = jnp.dot(f, wb_ref[...], preferred_element_type=jnp.float32) * sb


def _xw2(pA, pB, deg, hdA, hdB, bA, bB, wa, wb, ha, hb):
    return pl.pallas_call(
        functools.partial(_xw2_body, hdA, hdB, ha, hb),
        grid=(_NPAD // _BLK,),
        in_specs=[
            pl.BlockSpec((_NC, _BLK, _D), lambda i: (0, i, 0)),
            pl.BlockSpec((_NC, _BLK, _D), lambda i: (0, i, 0)),
            pl.BlockSpec((_BLK, 8), lambda i: (i, 0)),
            pl.BlockSpec((1, _D), lambda i: (0, 0)),
            pl.BlockSpec((1, _D), lambda i: (0, 0)),
            pl.BlockSpec((_D, _D), lambda i: (0, 0)),
            pl.BlockSpec((_D, _D), lambda i: (0, 0)),
        ],
        out_specs=[
            pl.BlockSpec((_BLK, _D), lambda i: (i, 0)),
            pl.BlockSpec((_BLK, _D), lambda i: (i, 0)),
        ],
        out_shape=[
            jax.ShapeDtypeStruct((_NPAD, _D), jnp.float32),
            jax.ShapeDtypeStruct((_NPAD, _D), jnp.float32),
        ],
    )(pA, pB, deg, bA, bB, wa, wb)


def _readout_body(p0_ref, p1_ref, p2_ref, p3_ref, deg_ref,
                  b0_ref, b1_ref, b2_ref, b3_ref,
                  wp1_ref, bp1_ref, wp2_ref, bp2_ref, o_ref, s_ref):
    i = pl.program_id(0)
    fg = jnp.maximum(
        (p0_ref[0] + p0_ref[1]) * _rs(deg_ref[:, 1:2]) + b0_ref[...]
        + (p2_ref[0] + p2_ref[1]) * _rs(deg_ref[:, 5:6]) + b2_ref[...], 0.0)
    fp = jnp.maximum(
        (p1_ref[0] + p1_ref[1]) * _rs(deg_ref[:, 3:4]) + b1_ref[...]
        + (p3_ref[0] + p3_ref[1]) * _rs(deg_ref[:, 7:8]) + b3_ref[...], 0.0)
    rows = i * _BLK + lax.broadcasted_iota(jnp.int32, (_BLK, 1), 0)
    contrib = jnp.where(rows < _N, fg + fp, 0.0)

    @pl.when(i == 0)
    def _():
        s_ref[...] = jnp.zeros_like(s_ref)

    s_ref[...] += contrib

    @pl.when(i == _NPAD // _BLK - 1)
    def _():
        g = jnp.sum(s_ref[...], axis=0, keepdims=True) * (1.0 / (2 * _N))
        h = jnp.maximum(
            jnp.dot(g, wp1_ref[...], preferred_element_type=jnp.float32)
            + bp1_ref[...], 0.0)
        out = jnp.dot(h, wp2_ref[...], preferred_element_type=jnp.float32)
        o_ref[...] = jnp.broadcast_to(out + bp2_ref[...], (8, _D))


def _readout(p0, p1, p2, p3, deg, b0, b1, b2, b3, wp1, bp1, wp2, bp2):
    part = pl.BlockSpec((_NC, _BLK, _D), lambda i: (0, i, 0))
    fixed_d = pl.BlockSpec((1, _D), lambda i: (0, 0))
    return pl.pallas_call(
        _readout_body,
        grid=(_NPAD // _BLK,),
        in_specs=[
            part, part, part, part,
            pl.BlockSpec((_BLK, 8), lambda i: (i, 0)),
            fixed_d, fixed_d, fixed_d, fixed_d,
            pl.BlockSpec((_D, _H), lambda i: (0, 0)),
            pl.BlockSpec((1, _H), lambda i: (0, 0)),
            pl.BlockSpec((_H, _D), lambda i: (0, 0)),
            fixed_d,
        ],
        out_specs=pl.BlockSpec((8, _D), lambda i: (0, 0)),
        out_shape=jax.ShapeDtypeStruct((8, _D), jnp.float32),
        scratch_shapes=[pltpu.VMEM((_BLK, _D), jnp.float32)],
    )(p0, p1, p2, p3, deg, b0, b1, b2, b3, wp1, bp1, wp2, bp2)


@functools.cache
def _mesh():
    return plsc.VectorSubcoreMesh(core_axis_name="c", subcore_axis_name="s")


def _deg_call(idx_hbm_arr):
    ones = jnp.ones((_NB,), jnp.float32)
    z1 = jnp.zeros((_ROWS_PER_TILE,), jnp.float32)
    return _build_deg()(idx_hbm_arr, ones, z1)


@functools.cache
def _build_deg():
    return functools.partial(
        pl.kernel,
        out_type=jax.ShapeDtypeStruct((_NC, 8, _NPAD), jnp.float32),
        mesh=_mesh(),
        scratch_types=[
            pltpu.VMEM((_NB,), jnp.int32),
            pltpu.VMEM((_NB,), jnp.float32),
            pltpu.VMEM((_ROWS_PER_TILE,), jnp.float32),
        ] + [pltpu.VMEM_SHARED((_NPAD,), jnp.float32) for _ in range(8)],
    )(_deg_body)


def _deg_body(idx_hbm, ones_hbm, z_hbm, out_hbm, idx_v, ones_v, zbuf_v, *accs):
    cid = lax.axis_index("c")
    sid = lax.axis_index("s")
    wid = cid * _NS + sid

    pltpu.sync_copy(ones_hbm, ones_v)
    pltpu.sync_copy(z_hbm, zbuf_v)
    sl = pl.ds(sid * _ROWS_PER_TILE, _ROWS_PER_TILE)
    for h in range(8):
        pltpu.sync_copy(zbuf_v, accs[h].at[sl])
    plsc.subcore_barrier()

    for h in range(8):
        def step(j, carry, h=h):
            base = wid * (_EBLK * _NB) + j * _NB
            pltpu.sync_copy(idx_hbm.at[h, pl.ds(base, _NB)], idx_v)
            pltpu.sync_copy(ones_v, accs[h].at[idx_v], add=True)
            return carry

        lax.fori_loop(0, _EBLK, step, 0)

    plsc.subcore_barrier()
    for h in range(8):
        pltpu.sync_copy(accs[h].at[sl], out_hbm.at[cid, h, sl])


def _scatter_call(y0, y1, y2, y3, srcs, dsts):
    zD = jnp.zeros((_NB, _D), jnp.float32)
    return _build_scatter()(y0, y1, y2, y3, srcs, dsts, zD)


@functools.cache
def _build_scatter():
    return functools.partial(
        pl.kernel,
        out_type=jax.ShapeDtypeStruct((4, _NC, _NPAD, _D), jnp.float32),
        mesh=_mesh(),
        scratch_types=[
            pltpu.VMEM((_NB,), jnp.int32),
            pltpu.VMEM((_NB,), jnp.int32),
            pltpu.VMEM((_NB,), jnp.int32),
            pltpu.VMEM((_NB,), jnp.int32),
            pltpu.VMEM((_NB, _D), jnp.float32),
            pltpu.VMEM((_NB, _D), jnp.float32),
            pltpu.VMEM((_NB, _D), jnp.float32),
            pltpu.VMEM_SHARED((_APAD, _D), jnp.float32),
            pltpu.SemaphoreType.DMA,
            pltpu.SemaphoreType.DMA,
        ],
    )(_scatter_body)


def _scatter_body(y0, y1, y2, y3, src_hbm, dst_hbm, zD_hbm, out_hbm,
                  s0, s1, d0, d1, rows0, rows1, zbuf_v, acc_sh, sem0, sem1):
    cid = lax.axis_index("c")
    sid = lax.axis_index("s")
    wid = cid * _NS + sid
    ys = (y0, y1, y2, y3)

    pltpu.sync_copy(zD_hbm, zbuf_v)

    for r in range(4):
        for k, (off, sz) in enumerate(_ZCHUNKS):
            pltpu.sync_copy(
                zbuf_v.at[pl.ds(0, sz)],
                acc_sh.at[pl.ds(sid * _ART + off, sz)])
        plsc.subcore_barrier()

        tb = wid * (_EBLK * _NB)
        pltpu.sync_copy(src_hbm.at[r, pl.ds(tb, _NB)], s0)
        pltpu.async_copy(ys[r].at[s0], rows0, sem0)

        def pair(jj, carry, r=r, tb=tb):
            b0 = tb + (2 * jj) * _NB
            b1 = b0 + _NB
            pltpu.sync_copy(src_hbm.at[r, pl.ds(b1, _NB)], s1)
            pltpu.async_copy(ys[r].at[s1], rows1, sem1)
            pltpu.make_async_copy(ys[r].at[s0], rows0, sem0).wait()
            pltpu.sync_copy(dst_hbm.at[r, pl.ds(b0, _NB)], d0)
            pltpu.sync_copy(rows0, acc_sh.at[d0], add=True)

            @pl.when(jj < _EBLK // 2 - 1)
            def _():
                b2 = b0 + 2 * _NB
                pltpu.sync_copy(src_hbm.at[r, pl.ds(b2, _NB)], s0)
                pltpu.async_copy(ys[r].at[s0], rows0, sem0)

            pltpu.make_async_copy(ys[r].at[s1], rows1, sem1).wait()
            pltpu.sync_copy(dst_hbm.at[r, pl.ds(b1, _NB)], d1)
            pltpu.sync_copy(rows1, acc_sh.at[d1], add=True)
            return carry

        lax.fori_loop(0, _EBLK // 2, pair, 0)
        plsc.subcore_barrier()
        pltpu.sync_copy(
            acc_sh.at[pl.ds(sid * _ART, _ART)],
            out_hbm.at[r, cid, pl.ds(sid * _ART, _ART)])


def _enc_fold(ep):
    c = 1.0 / math.sqrt(1.0 + 1e-3)
    c1 = ep['g1'] * c
    d1 = ep['b1'] * c1 + ep['be1']
    c2 = ep['g2'] * c
    d2 = ep['b2'] * c2 + ep['be2']
    return (ep['W1'], c1.reshape(1, -1), d1.reshape(1, -1),
            ep['W2'], c2.reshape(1, -1), d2.reshape(1, -1))


def kernel(x_gene, x_protein, ei_gene_to_gene, ei_gene_to_protein,
           ei_protein_to_gene, ei_protein_to_protein, params):
    p = params

    pad_idx = _N + (jnp.arange(_EPAD - _E, dtype=jnp.int32) % (_APAD - _N))
    pad_pair = jnp.stack([pad_idx, pad_idx])

    def pad_e(ei):
        return jnp.concatenate([ei.astype(jnp.int32), pad_pair], axis=1)

    eis = [pad_e(e) for e in (ei_gene_to_gene, ei_gene_to_protein,
                              ei_protein_to_gene, ei_protein_to_protein)]
    srcs = jnp.stack([e[0] for e in eis])
    dsts = jnp.stack([e[1] for e in eis])
    hist_idx = jnp.stack([eis[r][e] for r in range(4) for e in range(2)])

    degp = _deg_call(hist_idx)
    deg = (degp[0] + degp[1]).T

    xg = jnp.pad(x_gene, ((0, _NPAD - _N), (0, 0)))
    xp = jnp.pad(x_protein, ((0, _NPAD - _N), (0, 0)))
    fg = _encode(xg, *_enc_fold(p['enc_gene']))
    fp = _encode(xp, *_enc_fold(p['enc_protein']))

    def b2d(b):
        return b.reshape(1, _D)

    y0, y1 = _xw1(fg, p['gnn0_gene_to_gene']['W'], p['gnn0_gene_to_protein']['W'], deg, 0, 2)
    y2, y3 = _xw1(fp, p['gnn0_protein_to_gene']['W'], p['gnn0_protein_to_protein']['W'], deg, 4, 6)
    acc = _scatter_call(y0, y1, y2, y3, srcs, dsts)

    y0, y1 = _xw2(acc[0], acc[2], deg, 1, 5,
                  b2d(p['gnn0_gene_to_gene']['b']), b2d(p['gnn0_protein_to_gene']['b']),
                  p['gnn1_gene_to_gene']['W'], p['gnn1_gene_to_protein']['W'], 0, 2)
    y2, y3 = _xw2(acc[1], acc[3], deg, 3, 7,
                  b2d(p['gnn0_gene_to_protein']['b']), b2d(p['gnn0_protein_to_protein']['b']),
                  p['gnn1_protein_to_gene']['W'], p['gnn1_protein_to_protein']['W'], 4, 6)
    acc = _scatter_call(y0, y1, y2, y3, srcs, dsts)

    pp = p['pred']
    out = _readout(
        acc[0], acc[1], acc[2], acc[3], deg,
        b2d(p['gnn1_gene_to_gene']['b']), b2d(p['gnn1_gene_to_protein']['b']),
        b2d(p['gnn1_protein_to_gene']['b']), b2d(p['gnn1_protein_to_protein']['b']),
        pp['W1'], pp['b1'].reshape(1, _H),
        jnp.pad(pp['W2'], ((0, 0), (0, _D - 1))),
        jnp.broadcast_to(pp['b2'].reshape(1, 1), (1, _D)))
    return out[0, 0:1]


# speedup vs baseline: 12.7220x; 1.0258x over previous
import functools

import jax
import jax.numpy as jnp
from jax import lax
from jax.experimental import pallas as pl
from jax.experimental.pallas import tpu as pltpu
from jax.experimental.pallas import tpu_sc as plsc

_N = 10000
_NPAD = 10240
_E = 160000
_EPAD = 163840
_NB = 128
_EBLK = 40
_PAD_NODE = 10050
_D = 128
_H = 256
_BLK = 256
_NC = 2
_NS = 16
_ROWS_PER_TILE = _NPAD // _NS
_APAD = 10112
_ART = _APAD // _NS
_ZCHUNKS = [(0, 128), (128, 128), (256, 128), (384, 128), (512, 120)]


def _rs(x):
    return lax.rsqrt(jnp.maximum(x, 1.0))


def _enc_body(x_ref, w1_ref, b1_ref, g1_ref, be1_ref,
              w2_ref, b2_ref, g2_ref, be2_ref, o_ref):
    s = jnp.sqrt(jnp.float32(1.0 + 1e-3))
    h = jnp.dot(x_ref[...], w1_ref[...], preferred_element_type=jnp.float32)
    h = jnp.maximum((h + b1_ref[...]) / s * g1_ref[...] + be1_ref[...], 0.0)
    o = jnp.dot(h, w2_ref[...], preferred_element_type=jnp.float32)
    o_ref[...] = (o + b2_ref[...]) / s * g2_ref[...] + be2_ref[...]


def _encode(x, w1, b1, g1, be1, w2, b2, g2, be2):
    din = x.shape[1]
    return pl.pallas_call(
        _enc_body,
        grid=(_NPAD // _BLK,),
        in_specs=[
            pl.BlockSpec((_BLK, din), lambda i: (i, 0)),
            pl.BlockSpec((din, _H), lambda i: (0, 0)),
            pl.BlockSpec((1, _H), lambda i: (0, 0)),
            pl.BlockSpec((1, _H), lambda i: (0, 0)),
            pl.BlockSpec((1, _H), lambda i: (0, 0)),
            pl.BlockSpec((_H, _D), lambda i: (0, 0)),
            pl.BlockSpec((1, _D), lambda i: (0, 0)),
            pl.BlockSpec((1, _D), lambda i: (0, 0)),
            pl.BlockSpec((1, _D), lambda i: (0, 0)),
        ],
        out_specs=pl.BlockSpec((_BLK, _D), lambda i: (i, 0)),
        out_shape=jax.ShapeDtypeStruct((_NPAD, _D), jnp.float32),
    )(x, w1, b1, g1, be1, w2, b2, g2, be2)


def _xw1_body(ha, hb, f_ref, wa_ref, wb_ref, deg_ref, ya_ref, yb_ref):
    f = f_ref[...]
    sa = _rs(deg_ref[:, ha:ha + 1])
    sb = _rs(deg_ref[:, hb:hb + 1])
    ya_ref[...] = jnp.dot(f, wa_ref[...], preferred_element_type=jnp.float32) * sa
    yb_ref[...] = jnp.dot(f, wb_ref[...], preferred_element_type=jnp.float32) * sb


def _xw1(f, wa, wb, deg, ha, hb):
    return pl.pallas_call(
        functools.partial(_xw1_body, ha, hb),
        grid=(_NPAD // _BLK,),
        in_specs=[
            pl.BlockSpec((_BLK, _D), lambda i: (i, 0)),
            pl.BlockSpec((_D, _D), lambda i: (0, 0)),
            pl.BlockSpec((_D, _D), lambda i: (0, 0)),
            pl.BlockSpec((_BLK, 8), lambda i: (i, 0)),
        ],
        out_specs=[
            pl.BlockSpec((_BLK, _D), lambda i: (i, 0)),
            pl.BlockSpec((_BLK, _D), lambda i: (i, 0)),
        ],
        out_shape=[
            jax.ShapeDtypeStruct((_NPAD, _D), jnp.float32),
            jax.ShapeDtypeStruct((_NPAD, _D), jnp.float32),
        ],
    )(f, wa, wb, deg)


def _xw2_body(hdA, hdB, ha, hb, pA_ref, pB_ref, deg_ref, bA_ref, bB_ref,
              wa_ref, wb_ref, ya_ref, yb_ref):
    gA = (pA_ref[0] + pA_ref[1]) * _rs(deg_ref[:, hdA:hdA + 1]) + bA_ref[...]
    gB = (pB_ref[0] + pB_ref[1]) * _rs(deg_ref[:, hdB:hdB + 1]) + bB_ref[...]
    f = jnp.maximum(gA + gB, 0.0)
    sa = _rs(deg_ref[:, ha:ha + 1])
    sb = _rs(deg_ref[:, hb:hb + 1])
    ya_ref[...] = jnp.dot(f, wa_ref[...], preferred_element_type=jnp.float32) * sa
    yb_ref[...] = jnp.dot(f, wb_ref[...], preferred_element_type=jnp.float32) * sb


def _xw2(pA, pB, deg, hdA, hdB, bA, bB, wa, wb, ha, hb):
    return pl.pallas_call(
        functools.partial(_xw2_body, hdA, hdB, ha, hb),
        grid=(_NPAD // _BLK,),
        in_specs=[
            pl.BlockSpec((_NC, _BLK, _D), lambda i: (0, i, 0)),
            pl.BlockSpec((_NC, _BLK, _D), lambda i: (0, i, 0)),
            pl.BlockSpec((_BLK, 8), lambda i: (i, 0)),
            pl.BlockSpec((1, _D), lambda i: (0, 0)),
            pl.BlockSpec((1, _D), lambda i: (0, 0)),
            pl.BlockSpec((_D, _D), lambda i: (0, 0)),
            pl.BlockSpec((_D, _D), lambda i: (0, 0)),
        ],
        out_specs=[
            pl.BlockSpec((_BLK, _D), lambda i: (i, 0)),
            pl.BlockSpec((_BLK, _D), lambda i: (i, 0)),
        ],
        out_shape=[
            jax.ShapeDtypeStruct((_NPAD, _D), jnp.float32),
            jax.ShapeDtypeStruct((_NPAD, _D), jnp.float32),
        ],
    )(pA, pB, deg, bA, bB, wa, wb)


def _readout_body(p0_ref, p1_ref, p2_ref, p3_ref, deg_ref,
                  b0_ref, b1_ref, b2_ref, b3_ref,
                  wp1_ref, bp1_ref, wp2_ref, bp2_ref, o_ref, s_ref):
    i = pl.program_id(0)
    fg = jnp.maximum(
        (p0_ref[0] + p0_ref[1]) * _rs(deg_ref[:, 1:2]) + b0_ref[...]
        + (p2_ref[0] + p2_ref[1]) * _rs(deg_ref[:, 5:6]) + b2_ref[...], 0.0)
    fp = jnp.maximum(
        (p1_ref[0] + p1_ref[1]) * _rs(deg_ref[:, 3:4]) + b1_ref[...]
        + (p3_ref[0] + p3_ref[1]) * _rs(deg_ref[:, 7:8]) + b3_ref[...], 0.0)
    rows = i * _BLK + lax.broadcasted_iota(jnp.int32, (_BLK, 1), 0)
    contrib = jnp.where(rows < _N, fg + fp, 0.0)

    @pl.when(i == 0)
    def _():
        s_ref[...] = jnp.zeros_like(s_ref)

    s_ref[...] += contrib

    @pl.when(i == _NPAD // _BLK - 1)
    def _():
        g = jnp.sum(s_ref[...], axis=0, keepdims=True) * (1.0 / (2 * _N))
        h = jnp.maximum(
            jnp.dot(g, wp1_ref[...], preferred_element_type=jnp.float32)
            + bp1_ref[...], 0.0)
        out = jnp.dot(h, wp2_ref[...], preferred_element_type=jnp.float32)
        o_ref[...] = jnp.broadcast_to(out + bp2_ref[...], (8, _D))


def _readout(p0, p1, p2, p3, deg, b0, b1, b2, b3, wp1, bp1, wp2, bp2):
    part = pl.BlockSpec((_NC, _BLK, _D), lambda i: (0, i, 0))
    fixed_d = pl.BlockSpec((1, _D), lambda i: (0, 0))
    return pl.pallas_call(
        _readout_body,
        grid=(_NPAD // _BLK,),
        in_specs=[
            part, part, part, part,
            pl.BlockSpec((_BLK, 8), lambda i: (i, 0)),
            fixed_d, fixed_d, fixed_d, fixed_d,
            pl.BlockSpec((_D, _H), lambda i: (0, 0)),
            pl.BlockSpec((1, _H), lambda i: (0, 0)),
            pl.BlockSpec((_H, _D), lambda i: (0, 0)),
            fixed_d,
        ],
        out_specs=pl.BlockSpec((8, _D), lambda i: (0, 0)),
        out_shape=jax.ShapeDtypeStruct((8, _D), jnp.float32),
        scratch_shapes=[pltpu.VMEM((_BLK, _D), jnp.float32)],
    )(p0, p1, p2, p3, deg, b0, b1, b2, b3, wp1, bp1, wp2, bp2)


@functools.cache
def _mesh():
    return plsc.VectorSubcoreMesh(core_axis_name="c", subcore_axis_name="s")


def _deg_call(idx_hbm_arr):
    ones = jnp.ones((_NB,), jnp.float32)
    z1 = jnp.zeros((_ROWS_PER_TILE,), jnp.float32)
    return _build_deg()(idx_hbm_arr, ones, z1)


@functools.cache
def _build_deg():
    return functools.partial(
        pl.kernel,
        out_type=jax.ShapeDtypeStruct((_NC, 8, _NPAD), jnp.float32),
        mesh=_mesh(),
        scratch_types=[
            pltpu.VMEM((_NB,), jnp.int32),
            pltpu.VMEM((_NB,), jnp.int32),
            pltpu.VMEM((_NB,), jnp.float32),
            pltpu.VMEM((_ROWS_PER_TILE,), jnp.float32),
            pltpu.SemaphoreType.DMA,
            pltpu.SemaphoreType.DMA,
        ] + [pltpu.VMEM_SHARED((_NPAD,), jnp.float32) for _ in range(8)],
    )(_deg_body)


def _deg_body(idx_hbm, ones_hbm, z_hbm, out_hbm, i0, i1, ones_v, zbuf_v,
              sem0, sem1, *accs):
    cid = lax.axis_index("c")
    sid = lax.axis_index("s")
    wid = cid * _NS + sid

    pltpu.sync_copy(ones_hbm, ones_v)
    pltpu.sync_copy(z_hbm, zbuf_v)
    sl = pl.ds(sid * _ROWS_PER_TILE, _ROWS_PER_TILE)
    for h in range(8):
        pltpu.sync_copy(zbuf_v, accs[h].at[sl])
    plsc.subcore_barrier()

    tb = wid * (_EBLK * _NB)
    for h in range(8):
        acc = accs[h]
        pltpu.sync_copy(idx_hbm.at[h, pl.ds(tb, _NB)], i0)
        pltpu.async_copy(ones_v, acc.at[i0], sem0, add=True)

        def pair(jj, carry, h=h, acc=acc):
            b1 = tb + (2 * jj + 1) * _NB
            pltpu.sync_copy(idx_hbm.at[h, pl.ds(b1, _NB)], i1)
            pltpu.async_copy(ones_v, acc.at[i1], sem1, add=True)
            pltpu.make_async_copy(ones_v, acc.at[i0], sem0).wait()

            @pl.when(jj < _EBLK // 2 - 1)
            def _():
                b2 = tb + (2 * jj + 2) * _NB
                pltpu.sync_copy(idx_hbm.at[h, pl.ds(b2, _NB)], i0)
                pltpu.async_copy(ones_v, acc.at[i0], sem0, add=True)

            pltpu.make_async_copy(ones_v, acc.at[i1], sem1).wait()
            return carry

        lax.fori_loop(0, _EBLK // 2, pair, 0)

    plsc.subcore_barrier()
    for h in range(8):
        pltpu.sync_copy(accs[h].at[sl], out_hbm.at[cid, h, sl])


def _scatter_call(y0, y1, y2, y3, srcs, dsts):
    zD = jnp.zeros((_NB, _D), jnp.float32)
    return _build_scatter()(y0, y1, y2, y3, srcs, dsts, zD)


@functools.cache
def _build_scatter():
    return functools.partial(
        pl.kernel,
        out_type=jax.ShapeDtypeStruct((4, _NC, _NPAD, _D), jnp.float32),
        mesh=_mesh(),
        scratch_types=[
            pltpu.VMEM((_NB,), jnp.int32),
            pltpu.VMEM((_NB,), jnp.int32),
            pltpu.VMEM((_NB,), jnp.int32),
            pltpu.VMEM((_NB,), jnp.int32),
            pltpu.VMEM((_NB, _D), jnp.float32),
            pltpu.VMEM((_NB, _D), jnp.float32),
            pltpu.VMEM((_NB, _D), jnp.float32),
            pltpu.VMEM_SHARED((_APAD, _D), jnp.float32),
            pltpu.SemaphoreType.DMA,
            pltpu.SemaphoreType.DMA,
        ],
    )(_scatter_body)


def _scatter_body(y0, y1, y2, y3, src_hbm, dst_hbm, zD_hbm, out_hbm,
                  s0, s1, d0, d1, rows0, rows1, zbuf_v, acc_sh, sem0, sem1):
    cid = lax.axis_index("c")
    sid = lax.axis_index("s")
    wid = cid * _NS + sid
    ys = (y0, y1, y2, y3)

    pltpu.sync_copy(zD_hbm, zbuf_v)

    for r in range(4):
        for k, (off, sz) in enumerate(_ZCHUNKS):
            pltpu.sync_copy(
                zbuf_v.at[pl.ds(0, sz)],
                acc_sh.at[pl.ds(sid * _ART + off, sz)])
        plsc.subcore_barrier()

        tb = wid * (_EBLK * _NB)
        pltpu.sync_copy(src_hbm.at[r, pl.ds(tb, _NB)], s0)
        pltpu.async_copy(ys[r].at[s0], rows0, sem0)

        def pair(jj, carry, r=r, tb=tb):
            b0 = tb + (2 * jj) * _NB
            b1 = b0 + _NB
            pltpu.sync_copy(src_hbm.at[r, pl.ds(b1, _NB)], s1)
            pltpu.async_copy(ys[r].at[s1], rows1, sem1)
            pltpu.make_async_copy(ys[r].at[s0], rows0, sem0).wait()
            pltpu.sync_copy(dst_hbm.at[r, pl.ds(b0, _NB)], d0)
            pltpu.sync_copy(rows0, acc_sh.at[d0], add=True)

            @pl.when(jj < _EBLK // 2 - 1)
            def _():
                b2 = b0 + 2 * _NB
                pltpu.sync_copy(src_hbm.at[r, pl.ds(b2, _NB)], s0)
                pltpu.async_copy(ys[r].at[s0], rows0, sem0)

            pltpu.make_async_copy(ys[r].at[s1], rows1, sem1).wait()
            pltpu.sync_copy(dst_hbm.at[r, pl.ds(b1, _NB)], d1)
            pltpu.sync_copy(rows1, acc_sh.at[d1], add=True)
            return carry

        lax.fori_loop(0, _EBLK // 2, pair, 0)
        plsc.subcore_barrier()
        pltpu.sync_copy(
            acc_sh.at[pl.ds(sid * _ART, _ART)],
            out_hbm.at[r, cid, pl.ds(sid * _ART, _ART)])


def _enc_fold(ep):
    r = lambda v: v.reshape(1, -1)
    return (ep['W1'], r(ep['b1']), r(ep['g1']), r(ep['be1']),
            ep['W2'], r(ep['b2']), r(ep['g2']), r(ep['be2']))


def kernel(x_gene, x_protein, ei_gene_to_gene, ei_gene_to_protein,
           ei_protein_to_gene, ei_protein_to_protein, params):
    p = params

    pad_idx = _N + (jnp.arange(_EPAD - _E, dtype=jnp.int32) % (_APAD - _N))
    pad_pair = jnp.stack([pad_idx, pad_idx])

    def pad_e(ei):
        return jnp.concatenate([ei.astype(jnp.int32), pad_pair], axis=1)

    eis = [pad_e(e) for e in (ei_gene_to_gene, ei_gene_to_protein,
                              ei_protein_to_gene, ei_protein_to_protein)]
    srcs = jnp.stack([e[0] for e in eis])
    dsts = jnp.stack([e[1] for e in eis])
    hist_idx = jnp.stack([eis[r][e] for r in range(4) for e in range(2)])

    degp = _deg_call(hist_idx)
    deg = (degp[0] + degp[1]).T

    xg = jnp.pad(x_gene, ((0, _NPAD - _N), (0, 0)))
    xp = jnp.pad(x_protein, ((0, _NPAD - _N), (0, 0)))
    fg = _encode(xg, *_enc_fold(p['enc_gene']))
    fp = _encode(xp, *_enc_fold(p['enc_protein']))

    def b2d(b):
        return b.reshape(1, _D)

    y0, y1 = _xw1(fg, p['gnn0_gene_to_gene']['W'], p['gnn0_gene_to_protein']['W'], deg, 0, 2)
    y2, y3 = _xw1(fp, p['gnn0_protein_to_gene']['W'], p['gnn0_protein_to_protein']['W'], deg, 4, 6)
    acc = _scatter_call(y0, y1, y2, y3, srcs, dsts)

    y0, y1 = _xw2(acc[0], acc[2], deg, 1, 5,
                  b2d(p['gnn0_gene_to_gene']['b']), b2d(p['gnn0_protein_to_gene']['b']),
                  p['gnn1_gene_to_gene']['W'], p['gnn1_gene_to_protein']['W'], 0, 2)
    y2, y3 = _xw2(acc[1], acc[3], deg, 3, 7,
                  b2d(p['gnn0_gene_to_protein']['b']), b2d(p['gnn0_protein_to_protein']['b']),
                  p['gnn1_protein_to_gene']['W'], p['gnn1_protein_to_protein']['W'], 4, 6)
    acc = _scatter_call(y0, y1, y2, y3, srcs, dsts)

    pp = p['pred']
    out = _readout(
        acc[0], acc[1], acc[2], acc[3], deg,
        b2d(p['gnn1_gene_to_gene']['b']), b2d(p['gnn1_gene_to_protein']['b']),
        b2d(p['gnn1_protein_to_gene']['b']), b2d(p['gnn1_protein_to_protein']['b']),
        pp['W1'], pp['b1'].reshape(1, _H),
        jnp.pad(pp['W2'], ((0, 0), (0, _D - 1))),
        jnp.broadcast_to(pp['b2'].reshape(1, 1), (1, _D)))
    return out[0, 0:1]
